# baseline (speedup 1.0000x reference)
import functools

import jax
import jax.numpy as jnp
from jax import lax
from jax.experimental import pallas as pl
from jax.experimental.pallas import tpu as pltpu
from jax.experimental.pallas import tpu_sc as plsc

CTX_DIM = 512
N_CLS_CTX = 4
N_PRE = 5
TOK_LEN = 77
N_SUF = TOK_LEN - N_PRE - N_CLS_CTX
LANES = 16
CHUNK = 8


def _sc_prompts(table3d, label, token_prefix, token_suffix, b):
    info = plsc.get_sparse_core_info()
    num_workers = info.num_cores * info.num_subcores
    assert b % num_workers == 0
    bpw = b // num_workers
    assert bpw % CHUNK == 0
    n_chunks = bpw // CHUNK
    lane_steps = CTX_DIM // LANES

    mesh = plsc.VectorSubcoreMesh(core_axis_name="c", subcore_axis_name="s")

    @functools.partial(
        pl.kernel,
        mesh=mesh,
        out_type=jax.ShapeDtypeStruct((b, TOK_LEN, CTX_DIM), jnp.float32),
        scratch_types=[
            pltpu.VMEM((bpw,), jnp.int32),
            pltpu.VMEM((CHUNK, N_CLS_CTX, CTX_DIM), jnp.float32),
            pltpu.VMEM((CHUNK, N_CLS_CTX, CTX_DIM), jnp.float32),
            pltpu.VMEM((TOK_LEN, CTX_DIM), jnp.float32),
            pltpu.VMEM((N_PRE, CTX_DIM), jnp.float32),
            pltpu.VMEM((N_SUF, CTX_DIM), jnp.float32),
            pltpu.SemaphoreType.DMA,
            pltpu.SemaphoreType.DMA,
            pltpu.SemaphoreType.DMA,
            pltpu.SemaphoreType.DMA,
        ],
    )
    def body(table_hbm, idx_hbm, pre_hbm, suf_hbm, out_hbm,
             idx_v, rga, rgb, blk, pre_v, suf_v,
             gsema, gsemb, osem, tsem):
        wid = lax.axis_index("s") * info.num_cores + lax.axis_index("c")
        base = wid * bpw
        pltpu.sync_copy(idx_hbm.at[pl.ds(base, bpw)], idx_v)
        cp_p = pltpu.make_async_copy(pre_hbm.at[0], pre_v, tsem)
        cp_s = pltpu.make_async_copy(suf_hbm.at[0], suf_v, tsem)
        cp_p.start()
        cp_s.start()
        cp_p.wait()
        cp_s.wait()

        for r in range(N_PRE):
            for c in range(lane_steps):
                sl = pl.ds(c * LANES, LANES)
                blk[r, sl] = pre_v[r, sl]

        def fill_suffix(r, _):
            for c in range(lane_steps):
                sl = pl.ds(c * LANES, LANES)
                blk[9 + r, sl] = suf_v[r, sl]
            return _

        lax.fori_loop(0, N_SUF, fill_suffix, 0)

        gather_bufs = (rga, rgb)
        gather_sems = (gsema, gsemb)

        def start_gather(c):
            pltpu.make_async_copy(
                table_hbm.at[idx_v.at[pl.ds(c * CHUNK, CHUNK)]],
                gather_bufs[c % 2], gather_sems[c % 2]).start()

        start_gather(0)
        for c in range(n_chunks):
            rg = gather_bufs[c % 2]
            pltpu.make_async_copy(
                table_hbm.at[idx_v.at[pl.ds(c * CHUNK, CHUNK)]],
                rg, gather_sems[c % 2]).wait()
            if c + 1 < n_chunks:
                start_gather(c + 1)

            def do_element(t, _):
                j = c * CHUNK + t

                @pl.when(jnp.logical_or(t > 0, c > 0))
                def _wait_prev():
                    pltpu.make_async_copy(blk, out_hbm.at[base], osem).wait()

                for r in range(N_CLS_CTX):
                    for cc in range(lane_steps):
                        sl = pl.ds(cc * LANES, LANES)
                        blk[N_PRE + r, sl] = rg[t, r, sl]
                pltpu.make_async_copy(blk, out_hbm.at[base + j], osem).start()
                return _

            lax.fori_loop(0, CHUNK, do_element, 0)

        pltpu.make_async_copy(blk, out_hbm.at[base], osem).wait()

    return body(table3d, label, token_prefix, token_suffix)


def kernel(label, cls_ctx, token_prefix, token_suffix):
    b = label.shape[0]
    return _sc_prompts(cls_ctx, label.astype(jnp.int32), token_prefix,
                       token_suffix, b)


# speedup vs baseline: 1.0521x; 1.0521x over previous
import functools

import jax
import jax.numpy as jnp
from jax import lax
from jax.experimental import pallas as pl
from jax.experimental.pallas import tpu as pltpu
from jax.experimental.pallas import tpu_sc as plsc

CTX_DIM = 512
N_CLS_CTX = 4
N_PRE = 5
TOK_LEN = 77
N_SUF = TOK_LEN - N_PRE - N_CLS_CTX
LANES = 16
CHUNK = 8
TMPL_ROWS = 80


def _sc_prompts(table3d, label, template, b):
    info = plsc.get_sparse_core_info()
    num_workers = info.num_cores * info.num_subcores
    assert b % num_workers == 0
    bpw = b // num_workers
    assert bpw % CHUNK == 0 and CHUNK % 2 == 0
    n_chunks = bpw // CHUNK
    lane_steps = CTX_DIM // LANES

    mesh = plsc.VectorSubcoreMesh(core_axis_name="c", subcore_axis_name="s")

    @functools.partial(
        pl.kernel,
        mesh=mesh,
        out_type=jax.ShapeDtypeStruct((b, TOK_LEN, CTX_DIM), jnp.float32),
        scratch_types=[
            pltpu.VMEM((bpw,), jnp.int32),
            pltpu.VMEM((CHUNK, N_CLS_CTX, CTX_DIM), jnp.float32),
            pltpu.VMEM((CHUNK, N_CLS_CTX, CTX_DIM), jnp.float32),
            pltpu.VMEM((TOK_LEN, CTX_DIM), jnp.float32),
            pltpu.VMEM((TOK_LEN, CTX_DIM), jnp.float32),
            pltpu.VMEM((8, CTX_DIM), jnp.float32),
            pltpu.SemaphoreType.DMA,
            pltpu.SemaphoreType.DMA,
            pltpu.SemaphoreType.DMA,
            pltpu.SemaphoreType.DMA,
            pltpu.SemaphoreType.DMA,
        ],
    )
    def body(table_hbm, idx_hbm, tmpl_hbm, out_hbm,
             idx_v, rga, rgb, blk0, blk1, tail_v,
             gsema, gsemb, osem0, osem1, tsem):
        wid = lax.axis_index("s") * info.num_cores + lax.axis_index("c")
        base = wid * bpw
        pltpu.sync_copy(idx_hbm.at[pl.ds(base, bpw)], idx_v)
        head = [
            pltpu.make_async_copy(tmpl_hbm.at[pl.ds(0, 72)],
                                  blk.at[pl.ds(0, 72)], tsem)
            for blk in (blk0, blk1)
        ]
        tail = pltpu.make_async_copy(tmpl_hbm.at[pl.ds(72, 8)], tail_v, tsem)
        for cp in head + [tail]:
            cp.start()
        for cp in head + [tail]:
            cp.wait()
        for blk in (blk0, blk1):
            for r in range(72, TOK_LEN):
                for c in range(lane_steps):
                    sl = pl.ds(c * LANES, LANES)
                    blk[r, sl] = tail_v[r - 72, sl]

        gather_bufs = (rga, rgb)
        gather_sems = (gsema, gsemb)

        def start_gather(c):
            pltpu.make_async_copy(
                table_hbm.at[idx_v.at[pl.ds(c * CHUNK, CHUNK)]],
                gather_bufs[c % 2], gather_sems[c % 2]).start()

        start_gather(0)
        for c in range(n_chunks):
            rg = gather_bufs[c % 2]
            pltpu.make_async_copy(
                table_hbm.at[idx_v.at[pl.ds(c * CHUNK, CHUNK)]],
                rg, gather_sems[c % 2]).wait()
            if c + 1 < n_chunks:
                start_gather(c + 1)

            def do_pair(t2, _):
                j0 = c * CHUNK + 2 * t2
                j1 = j0 + 1

                @pl.when(j0 > 0)
                def _wait0():
                    pltpu.make_async_copy(blk0, out_hbm.at[base], osem0).wait()

                for r in range(N_CLS_CTX):
                    for cc in range(lane_steps):
                        sl = pl.ds(cc * LANES, LANES)
                        blk0[N_PRE + r, sl] = rg[2 * t2, r, sl]
                pltpu.make_async_copy(blk0, out_hbm.at[base + j0],
                                      osem0).start()

                @pl.when(j1 > 1)
                def _wait1():
                    pltpu.make_async_copy(blk1, out_hbm.at[base], osem1).wait()

                for r in range(N_CLS_CTX):
                    for cc in range(lane_steps):
                        sl = pl.ds(cc * LANES, LANES)
                        blk1[N_PRE + r, sl] = rg[2 * t2 + 1, r, sl]
                pltpu.make_async_copy(blk1, out_hbm.at[base + j1],
                                      osem1).start()
                return _

            lax.fori_loop(0, CHUNK // 2, do_pair, 0)

        pltpu.make_async_copy(blk0, out_hbm.at[base], osem0).wait()
        pltpu.make_async_copy(blk1, out_hbm.at[base], osem1).wait()

    return body(table3d, label, template)


def kernel(label, cls_ctx, token_prefix, token_suffix):
    b = label.shape[0]
    template = jnp.zeros((TMPL_ROWS, CTX_DIM), jnp.float32)
    template = template.at[0:N_PRE].set(token_prefix[0])
    template = template.at[N_PRE + N_CLS_CTX:TOK_LEN].set(token_suffix[0])
    return _sc_prompts(cls_ctx, label.astype(jnp.int32), template, b)
